# Initial kernel scaffold; baseline (speedup 1.0000x reference)
import jax
import jax.numpy as jnp
from jax.experimental import pallas as pl


def kernel(x, t, table, W1, b1, W2, b2):
    raise NotImplementedError("write your pallas kernel here")


# speedup vs baseline: 2.2445x; 2.2445x over previous
import functools

import jax
import jax.numpy as jnp
from jax import lax
from jax.experimental import pallas as pl
from jax.experimental.pallas import tpu as pltpu
from jax.experimental.pallas import tpu_sc as plsc

_NC = 2
_NS = 16
_NW = _NC * _NS
_LANES = 16
_CA = 128


def _bow_sums_sc(x, table):
    batch, seq = x.shape
    ca = _CA
    cb = seq - ca
    hid = table.shape[1]
    nh = hid // _LANES
    b_per_w = batch // _NW

    mesh = plsc.VectorSubcoreMesh(
        core_axis_name="c", subcore_axis_name="s",
        num_cores=_NC, num_subcores=_NS,
    )

    @functools.partial(
        pl.kernel,
        out_type=jax.ShapeDtypeStruct((batch, hid), jnp.float32),
        mesh=mesh,
        scratch_types=[
            pltpu.VMEM((b_per_w, seq), jnp.int32),
            pltpu.VMEM((3, ca, hid), jnp.float32),
            pltpu.VMEM((3, cb, hid), jnp.float32),
            pltpu.VMEM((b_per_w, hid), jnp.float32),
            pltpu.SemaphoreType.DMA,
            pltpu.SemaphoreType.DMA,
            pltpu.SemaphoreType.DMA,
            pltpu.SemaphoreType.DMA,
            pltpu.SemaphoreType.DMA,
            pltpu.SemaphoreType.DMA,
        ],
    )
    def bow_kernel(x_hbm, table_hbm, out_hbm, idx_v, rows_a, rows_b, bow_v,
                   sa0, sa1, sa2, sb0, sb1, sb2):
        wid = lax.axis_index("s") * _NC + lax.axis_index("c")
        pltpu.sync_copy(x_hbm.at[pl.ds(wid * b_per_w, b_per_w)], idx_v)
        sems_a = (sa0, sa1, sa2)
        sems_b = (sb0, sb1, sb2)

        def gather_a(s, k):
            return pltpu.make_async_copy(
                table_hbm.at[idx_v.at[s, pl.ds(0, ca)]],
                rows_a.at[k], sems_a[k],
            )

        def gather_b(s, k):
            return pltpu.make_async_copy(
                table_hbm.at[idx_v.at[s, pl.ds(ca, cb)]],
                rows_b.at[k], sems_b[k],
            )

        def accum_sample(k, accs):
            def body1(i, accs):
                new = list(accs)
                for h in range(nh):
                    new[h] = new[h] + rows_a[k, i, pl.ds(h * _LANES, _LANES)]
                for h in range(nh):
                    new[h] = new[h] + rows_b[k, i, pl.ds(h * _LANES, _LANES)]
                return tuple(new)

            def body2(i, accs):
                return tuple(
                    accs[h] + rows_a[k, i, pl.ds(h * _LANES, _LANES)]
                    for h in range(nh)
                )

            accs = lax.fori_loop(0, cb, body1, accs, unroll=2)
            return lax.fori_loop(cb, ca, body2, accs, unroll=4)

        def zeros():
            return tuple(jnp.zeros((_LANES,), jnp.float32) for _ in range(nh))

        def flush(b, accs):
            for h in range(nh):
                bow_v[b, pl.ds(h * _LANES, _LANES)] = accs[h]

        for s in range(2):
            gather_a(s, s).start()
            gather_b(s, s).start()
        n_trips = b_per_w // 3
        tail_base = 3 * n_trips

        def per_trip(p, _):
            s0 = 3 * p
            for k in range(3):
                s = s0 + k
                gather_a(s, k).wait()
                gather_b(s, k).wait()
                accs = accum_sample(k, zeros())
                flush(s, accs)

                @pl.when(s + 2 < b_per_w)
                def _():
                    gather_a(s + 2, (k + 2) % 3).start()
                    gather_b(s + 2, (k + 2) % 3).start()

            return 0

        lax.fori_loop(0, n_trips, per_trip, 0)
        for k in range(b_per_w - tail_base):
            s = tail_base + k
            gather_a(s, k).wait()
            gather_b(s, k).wait()
            accs = accum_sample(k, zeros())
            flush(s, accs)
        pltpu.sync_copy(bow_v, out_hbm.at[pl.ds(wid * b_per_w, b_per_w)])

    return bow_kernel(x, table)


def _mlp_tc(bow_sums, t, w1, b1, w2t, b2, inv_l):
    batch, hid = bow_sums.shape

    def mlp_body(bow_ref, t_ref, w1_ref, b1_ref, w2_ref, b2_ref,
                 logits_ref, loss_ref):
        bow = bow_ref[...] * inv_l
        h = jnp.maximum(
            jnp.dot(bow, w1_ref[...], preferred_element_type=jnp.float32)
            + b1_ref[...],
            0.0,
        )
        z = jnp.sum(h * w2_ref[...], axis=1) + b2_ref[0]
        logits_ref[...] = z
        per = (jnp.maximum(z, 0.0) - z * t_ref[...]
               + jnp.log1p(jnp.exp(-jnp.abs(z))))
        loss_ref[0] = jnp.sum(per) * (1.0 / batch)

    vmem = pl.BlockSpec(memory_space=pltpu.VMEM)
    smem = pl.BlockSpec(memory_space=pltpu.SMEM)
    return pl.pallas_call(
        mlp_body,
        in_specs=[vmem, vmem, vmem, vmem, vmem, smem],
        out_specs=[vmem, smem],
        out_shape=[
            jax.ShapeDtypeStruct((batch,), jnp.float32),
            jax.ShapeDtypeStruct((1,), jnp.float32),
        ],
    )(bow_sums, t, w1, b1, w2t, b2)


def kernel(x, t, table, W1, b1, W2, b2):
    batch, seq = x.shape
    bow_sums = _bow_sums_sc(x.astype(jnp.int32), table)
    logits, loss = _mlp_tc(
        bow_sums,
        t,
        W1,
        b1.reshape(1, -1),
        W2.reshape(1, -1),
        b2,
        1.0 / seq,
    )
    return (loss[0], logits)


# speedup vs baseline: 2.7755x; 1.2366x over previous
import functools

import jax
import jax.numpy as jnp
from jax import lax
from jax.experimental import pallas as pl
from jax.experimental.pallas import tpu as pltpu
from jax.experimental.pallas import tpu_sc as plsc

_NC = 2
_NS = 16
_NW = _NC * _NS
_LANES = 16


def _bow_sums_sc(x2, table):
    b2, chunk = x2.shape
    batch = b2 // 2
    hid = table.shape[1]
    nh = hid // _LANES
    b_per_w = batch // _NW
    rows_per_w = 2 * b_per_w

    mesh = plsc.VectorSubcoreMesh(
        core_axis_name="c", subcore_axis_name="s",
        num_cores=_NC, num_subcores=_NS,
    )

    @functools.partial(
        pl.kernel,
        out_type=jax.ShapeDtypeStruct((batch, hid), jnp.float32),
        mesh=mesh,
        scratch_types=[
            pltpu.VMEM((rows_per_w, chunk), jnp.int32),
            pltpu.VMEM((6, chunk, hid), jnp.float32),
            pltpu.VMEM((b_per_w, hid), jnp.float32),
            pltpu.SemaphoreType.DMA,
            pltpu.SemaphoreType.DMA,
            pltpu.SemaphoreType.DMA,
            pltpu.SemaphoreType.DMA,
            pltpu.SemaphoreType.DMA,
            pltpu.SemaphoreType.DMA,
        ],
    )
    def bow_kernel(x_hbm, table_hbm, out_hbm, idx_v, rows_v, bow_v,
                   sem0, sem1, sem2, sem3, sem4, sem5):
        wid = lax.axis_index("s") * _NC + lax.axis_index("c")
        row_base = wid * rows_per_w
        pltpu.sync_copy(x_hbm.at[pl.ds(row_base, rows_per_w)], idx_v)
        sems = (sem0, sem1, sem2, sem3, sem4, sem5)

        def gather(r, buf):
            return pltpu.make_async_copy(
                table_hbm.at[idx_v.at[r]], rows_v.at[buf], sems[buf]
            )

        def accum_pair(buf0, buf1, accs):
            def body(i, accs):
                new = list(accs)
                for buf in (buf0, buf1):
                    for h in range(nh):
                        new[h] = new[h] + rows_v[
                            buf, i, pl.ds(h * _LANES, _LANES)
                        ]
                return tuple(new)
            return lax.fori_loop(0, chunk, body, accs, unroll=2)

        def zeros():
            return tuple(jnp.zeros((_LANES,), jnp.float32) for _ in range(nh))

        def flush(b, accs):
            for h in range(nh):
                bow_v[b, pl.ds(h * _LANES, _LANES)] = accs[h]

        n_chunks = rows_per_w
        for c in range(5):
            gather(c, c).start()
        n_trips = b_per_w // 3
        tail_base = 3 * n_trips

        def per_trip(p, _):
            c0 = 6 * p
            gather(c0 + 5, 5).start()
            for s in range(3):
                b0, b1 = 2 * s, 2 * s + 1
                gather(c0 + b0, b0).wait()
                gather(c0 + b1, b1).wait()
                accs = accum_pair(b0, b1, zeros())
                flush(3 * p + s, accs)
                for buf in (b0, b1):
                    if buf < 5:
                        @pl.when(c0 + 6 + buf < n_chunks)
                        def _():
                            gather(c0 + 6 + buf, buf).start()

            return 0

        lax.fori_loop(0, n_trips, per_trip, 0)
        c0 = 6 * n_trips
        for s in range(b_per_w - tail_base):
            b0, b1 = 2 * s, 2 * s + 1
            gather(c0 + b0, b0).wait()
            gather(c0 + b1, b1).wait()
            accs = accum_pair(b0, b1, zeros())
            flush(tail_base + s, accs)
        pltpu.sync_copy(bow_v, out_hbm.at[pl.ds(wid * b_per_w, b_per_w)])

    return bow_kernel(x2, table)


def _mlp_tc(bow_sums, t, w1, b1, w2t, b2, inv_l):
    batch, hid = bow_sums.shape

    def mlp_body(bow_ref, t_ref, w1_ref, b1_ref, w2_ref, b2_ref,
                 logits_ref, loss_ref):
        bow = bow_ref[...] * inv_l
        h = jnp.maximum(
            jnp.dot(bow, w1_ref[...], preferred_element_type=jnp.float32)
            + b1_ref[...],
            0.0,
        )
        z = jnp.sum(h * w2_ref[...], axis=1) + b2_ref[0]
        logits_ref[...] = z
        per = (jnp.maximum(z, 0.0) - z * t_ref[...]
               + jnp.log1p(jnp.exp(-jnp.abs(z))))
        loss_ref[0] = jnp.sum(per) * (1.0 / batch)

    vmem = pl.BlockSpec(memory_space=pltpu.VMEM)
    smem = pl.BlockSpec(memory_space=pltpu.SMEM)
    return pl.pallas_call(
        mlp_body,
        in_specs=[vmem, vmem, vmem, vmem, vmem, smem],
        out_specs=[vmem, smem],
        out_shape=[
            jax.ShapeDtypeStruct((batch,), jnp.float32),
            jax.ShapeDtypeStruct((1,), jnp.float32),
        ],
    )(bow_sums, t, w1, b1, w2t, b2)


def kernel(x, t, table, W1, b1, W2, b2):
    batch, seq = x.shape
    bow_sums = _bow_sums_sc(
        x.astype(jnp.int32).reshape(batch * 2, seq // 2), table
    )
    logits, loss = _mlp_tc(
        bow_sums,
        t,
        W1,
        b1.reshape(1, -1),
        W2.reshape(1, -1),
        b2,
        1.0 / seq,
    )
    return (loss[0], logits)
